# baseline (speedup 1.0000x reference)
import functools

import jax
import jax.numpy as jnp
from jax import lax
from jax.experimental import pallas as pl
from jax.experimental.pallas import tpu as pltpu
from jax.experimental.pallas import tpu_sc as plsc

NUM_ITEMS = 1000000
DIM = 64
BATCH = 16384

_info = plsc.get_sparse_core_info()
NC, NS, L = _info.num_cores, _info.num_subcores, _info.num_lanes
NW = NC * NS

NREQ = 2 * BATCH
CW = 512
NCH = (NUM_ITEMS - 64) // CW
TAIL0 = NCH * CW
TAILB = NUM_ITEMS - 128
NBUK = 64
BCAP = 80
SLOTPT = NBUK * BCAP
NIDXR = SLOTPT // 128
NSLOT = NW * SLOTPT
INVPAD = NREQ + 128

_mesh = plsc.VectorSubcoreMesh(core_axis_name="c", subcore_axis_name="s")


@functools.partial(
    pl.kernel,
    mesh=_mesh,
    out_type=(jax.ShapeDtypeStruct((NSLOT * DIM,), jnp.float32),
              jax.ShapeDtypeStruct((INVPAD,), jnp.int32)),
    compiler_params=pltpu.CompilerParams(
        needs_layout_passes=False, use_tc_tiling_on_sc=True),
    scratch_types=[
        pltpu.VMEM((NREQ,), jnp.int32),
        pltpu.VMEM((SLOTPT,), jnp.int32),
        pltpu.VMEM((NIDXR, 128), jnp.int32),
        pltpu.VMEM((NBUK,), jnp.int32),
        pltpu.VMEM((NIDXR, 128), jnp.int32),
        pltpu.VMEM((2, DIM, CW), jnp.float32),
        pltpu.VMEM((2 * BCAP * DIM,), jnp.float32),
        pltpu.SMEM((4,), jnp.int32),
        pltpu.SemaphoreType.DMA((2,)),
        pltpu.SemaphoreType.DMA((2,)),
        pltpu.SemaphoreType.DMA,
    ],
)
def _sc_harvest(row_hbm, col_hbm, tab_hbm, tail_hbm, gath_hbm, inv_hbm,
                req_v, bk_item, bk_dst, bk_cnt, slotid_v, slab_v, st_v,
                cnt_s, csem, osem, isem):
    wid = lax.axis_index("s") * NC + lax.axis_index("c")
    lane = lax.iota(jnp.int32, L)
    ones = jnp.zeros((L,), jnp.int32) + 1

    cstart = (NCH * wid) // NW
    cend = (NCH * (wid + 1)) // NW
    lo_own = cstart * CW
    hi_own = jnp.where(wid == NW - 1, NUM_ITEMS, cend * CW)

    pltpu.async_copy(tab_hbm.at[:, pl.ds(cstart * CW, CW)], slab_v.at[0],
                     csem.at[0])
    pltpu.async_copy(tab_hbm.at[:, pl.ds((cstart + 1) * CW, CW)], slab_v.at[1],
                     csem.at[1])

    pltpu.sync_copy(row_hbm, req_v.at[pl.ds(0, BATCH)])
    pltpu.sync_copy(col_hbm, req_v.at[pl.ds(BATCH, BATCH)])

    def init_body(r, carry):
        for j in range(128 // L):
            c16 = r * 128 + j * L
            bk_dst[r, pl.ds(j * L, L)] = jnp.zeros((L,), jnp.int32) + NREQ
            slotid_v[r, pl.ds(j * L, L)] = wid * SLOTPT + c16 + lane
        return carry
    lax.fori_loop(0, NIDXR, init_body, 0)

    def zero_body(v, carry):
        bk_cnt[pl.ds(v * L, L)] = jnp.zeros((L,), jnp.int32)
        return carry
    lax.fori_loop(0, NBUK // L, zero_body, 0)
    cnt_s[0] = 0

    def bcast(vec, f_splat):
        return jnp.take_along_axis(vec, f_splat, axis=0,
                                   mode="promise_in_bounds")

    def route_one(v):
        x = req_v[pl.ds(v * L, L)]
        m = (x >= lo_own) & (x < hi_own)

        def cond(state):
            return jnp.any(state[0])

        def take(state):
            m_cur, _ = state
            f = plsc.all_reduce_ffs(m_cur)
            sel = lane == f
            item = bcast(x, f)
            buk = lax.shift_right_logical(item - lo_own, 9)
            pos = plsc.load_gather(bk_cnt, [buk])
            m0 = sel & (pos < BCAP)
            s = buk * BCAP + pos
            plsc.store_scatter(bk_item, [s], item, mask=m0)
            plsc.store_scatter(
                bk_dst, [lax.shift_right_logical(s, 7), s & 127],
                lane + v * L, mask=m0)
            plsc.addupdate_scatter(bk_cnt, [buk], ones, mask=m0)
            return (m_cur & jnp.logical_not(sel), 0)

        lax.while_loop(cond, take, (m, 0))

    def route_body(u, carry):
        for j in range(4):
            route_one(u * 4 + j)
        return carry

    lax.fori_loop(0, NREQ // L // 4, route_body, 0)

    for j0 in range(0, NIDXR, 8):
        inv_copies = []
        for j in range(j0, j0 + 8):
            inv_copies.append(pltpu.async_copy(
                slotid_v.at[j], inv_hbm.at[bk_dst.at[j]], isem))
        for c in inv_copies:
            c.wait()

    def process_bucket(buf, buk, origin):
        nb = plsc.load_gather(bk_cnt, [jnp.zeros((L,), jnp.int32) + buk])
        bc = cnt_s[0]
        sbuf = lax.rem(bc, 2)

        @pl.when(bc >= 2)
        def _():
            pltpu.make_async_copy(
                gath_hbm.at[pl.ds(0, BCAP * DIM)],
                st_v.at[pl.ds(0, BCAP * DIM)],
                osem.at[sbuf]).wait()

        def vreg_body(v, carry):
            pv = v * L + lane
            m = pv < nb

            @pl.when(jnp.any(m))
            def _():
                x = bk_item[pl.ds(buk * BCAP + v * L, L)]
                lcs = jnp.where(m, x - origin, 0)
                dv = jnp.zeros((L,), jnp.int32)
                pidx = sbuf * (BCAP * DIM) + pv * DIM
                for _ in range(DIM):
                    g = plsc.load_gather(slab_v.at[buf], [dv, lcs], mask=m)
                    plsc.store_scatter(st_v, [pidx], g, mask=m)
                    dv = dv + 1
                    pidx = pidx + 1
            return carry

        lax.fori_loop(0, BCAP // L, vreg_body, 0)
        pltpu.async_copy(
            st_v.at[pl.ds(sbuf * (BCAP * DIM), BCAP * DIM)],
            gath_hbm.at[pl.ds((wid * SLOTPT + buk * BCAP) * DIM, BCAP * DIM)],
            osem.at[sbuf])
        cnt_s[0] = bc + 1

    def fire_slab(k, buf):
        pltpu.async_copy(tab_hbm.at[:, pl.ds(k * CW, CW)], slab_v.at[buf],
                         csem.at[buf])

    def wait_slab(buf):
        pltpu.make_async_copy(
            tab_hbm.at[:, pl.ds(0, CW)], slab_v.at[buf],
            csem.at[buf]).wait()

    nslab = cend - cstart

    def slab_body(i, carry):
        buf = lax.rem(i, 2)
        wait_slab(buf)
        process_bucket(buf, i, (cstart + i) * CW)

        @pl.when(i + 2 < nslab)
        def _():
            fire_slab(cstart + i + 2, buf)
        return carry

    lax.fori_loop(0, nslab, slab_body, 0)

    @pl.when(wid == NW - 1)
    def _():
        pltpu.sync_copy(tail_hbm, slab_v.at[0, :, pl.ds(0, 128)])
        process_bucket(0, nslab, TAILB)

    total = cnt_s[0]

    def drain_body(s, carry):
        @pl.when(s < total)
        def _():
            pltpu.make_async_copy(
                gath_hbm.at[pl.ds(0, BCAP * DIM)],
                st_v.at[pl.ds(0, BCAP * DIM)],
                osem.at[lax.rem(total + s, 2)]).wait()
        return carry

    lax.fori_loop(0, 2, drain_body, 0)


@functools.partial(
    pl.kernel,
    mesh=_mesh,
    out_type=jax.ShapeDtypeStruct((BATCH,), jnp.float32),
    compiler_params=pltpu.CompilerParams(
        needs_layout_passes=False, use_tc_tiling_on_sc=False),
    scratch_types=[
        pltpu.VMEM((4, 128), jnp.int32),
        pltpu.VMEM((4, 128), jnp.int32),
        pltpu.VMEM((BATCH // NW, DIM), jnp.float32),
        pltpu.VMEM((BATCH // NW, DIM), jnp.float32),
        pltpu.VMEM((BATCH // NW,), jnp.float32),
        pltpu.SemaphoreType.DMA,
    ],
)
def _sc_dot(gath_hbm, inv_hbm, out_hbm, iv_r, iv_c, rows_v, cols_v, out_v,
            sem):
    wid = lax.axis_index("s") * NC + lax.axis_index("c")
    bpw = BATCH // NW
    base = wid * bpw

    pltpu.sync_copy(inv_hbm.at[pl.ds(wid * 4, 4)], iv_r)
    pltpu.sync_copy(inv_hbm.at[pl.ds(BATCH // 128 + wid * 4, 4)], iv_c)

    copies = []
    for j in range(4):
        copies.append(pltpu.async_copy(
            gath_hbm.at[iv_r.at[j]], rows_v.at[pl.ds(j * 128, 128)], sem))
        copies.append(pltpu.async_copy(
            gath_hbm.at[iv_c.at[j]], cols_v.at[pl.ds(j * 128, 128)], sem))
    for c in copies:
        c.wait()

    lane = lax.iota(jnp.int32, L)

    def group_body(g, carry):
        row_ids = g * L + lane
        acc = jnp.zeros((L,), jnp.float32)
        dcol = jnp.zeros((L,), jnp.int32)
        for _ in range(DIM):
            r = plsc.load_gather(rows_v, [row_ids, dcol])
            c = plsc.load_gather(cols_v, [row_ids, dcol])
            acc = acc + r * c
            dcol = dcol + 1
        out_v[pl.ds(g * L, L)] = acc
        return carry

    lax.fori_loop(0, bpw // L, group_body, 0)

    pltpu.sync_copy(out_v, out_hbm.at[pl.ds(base, bpw)])


def kernel(rowIndex, colIndex, outEmbs):
    tabT = outEmbs.T
    tailT = lax.slice(outEmbs, (TAILB, 0), (NUM_ITEMS, DIM)).T
    gath, inv = _sc_harvest(rowIndex.astype(jnp.int32),
                            colIndex.astype(jnp.int32), tabT, tailT)
    return _sc_dot(gath.reshape(NSLOT, DIM), inv.reshape(INVPAD // 128, 128))


# speedup vs baseline: 32.9532x; 32.9532x over previous
import functools

import jax
import jax.numpy as jnp
from jax import lax
from jax.experimental import pallas as pl
from jax.experimental.pallas import tpu as pltpu
from jax.experimental.pallas import tpu_sc as plsc

NUM_ITEMS = 1000000
DIM = 64
BATCH = 16384

_info = plsc.get_sparse_core_info()
NC, NS, L = _info.num_cores, _info.num_subcores, _info.num_lanes
NW = NC * NS

NREQ = 2 * BATCH
CW = 512
NCH = (NUM_ITEMS - 64) // CW
TAIL0 = NCH * CW
TAILB = NUM_ITEMS - 128
NBUK = 64
BCAP = 80
SLOTPT = NBUK * BCAP
NIDXR = SLOTPT // 128
NSLOT = NW * SLOTPT
INVPAD = NREQ + NSLOT

_mesh = plsc.VectorSubcoreMesh(core_axis_name="c", subcore_axis_name="s")


@functools.partial(
    pl.kernel,
    mesh=_mesh,
    out_type=(jax.ShapeDtypeStruct((NSLOT * DIM,), jnp.float32),
              jax.ShapeDtypeStruct((INVPAD,), jnp.int32)),
    compiler_params=pltpu.CompilerParams(
        needs_layout_passes=False, use_tc_tiling_on_sc=True),
    scratch_types=[
        pltpu.VMEM((NREQ,), jnp.int32),
        pltpu.VMEM((SLOTPT,), jnp.int32),
        pltpu.VMEM((NIDXR, 128), jnp.int32),
        pltpu.VMEM((NBUK,), jnp.int32),
        pltpu.VMEM((NIDXR, 128), jnp.int32),
        pltpu.VMEM((2, DIM, CW), jnp.float32),
        pltpu.VMEM((2 * BCAP * DIM,), jnp.float32),
        pltpu.SMEM((4,), jnp.int32),
        pltpu.SemaphoreType.DMA((2,)),
        pltpu.SemaphoreType.DMA((2,)),
        pltpu.SemaphoreType.DMA,
    ],
)
def _sc_harvest(row_hbm, col_hbm, tab_hbm, tail_hbm, gath_hbm, inv_hbm,
                req_v, bk_item, bk_dst, bk_cnt, slotid_v, slab_v, st_v,
                cnt_s, csem, osem, isem):
    wid = lax.axis_index("s") * NC + lax.axis_index("c")
    lane = lax.iota(jnp.int32, L)
    ones = jnp.zeros((L,), jnp.int32) + 1

    cstart = (NCH * wid) // NW
    cend = (NCH * (wid + 1)) // NW
    lo_own = cstart * CW
    hi_own = jnp.where(wid == NW - 1, NUM_ITEMS, cend * CW)

    pltpu.async_copy(tab_hbm.at[:, pl.ds(cstart * CW, CW)], slab_v.at[0],
                     csem.at[0])
    pltpu.async_copy(tab_hbm.at[:, pl.ds((cstart + 1) * CW, CW)], slab_v.at[1],
                     csem.at[1])

    pltpu.sync_copy(row_hbm, req_v.at[pl.ds(0, BATCH)])
    pltpu.sync_copy(col_hbm, req_v.at[pl.ds(BATCH, BATCH)])

    def init_body(r, carry):
        for j in range(128 // L):
            c16 = r * 128 + j * L
            gslot = wid * SLOTPT + c16 + lane
            bk_dst[r, pl.ds(j * L, L)] = NREQ + gslot
            slotid_v[r, pl.ds(j * L, L)] = gslot
        return carry
    lax.fori_loop(0, NIDXR, init_body, 0)

    def zero_body(v, carry):
        bk_cnt[pl.ds(v * L, L)] = jnp.zeros((L,), jnp.int32)
        return carry
    lax.fori_loop(0, NBUK // L, zero_body, 0)
    cnt_s[0] = 0

    def bcast(vec, f_splat):
        return jnp.take_along_axis(vec, f_splat, axis=0,
                                   mode="promise_in_bounds")

    def route_one(v):
        x = req_v[pl.ds(v * L, L)]
        m = (x >= lo_own) & (x < hi_own)

        def cond(state):
            return jnp.any(state[0])

        def take(state):
            m_cur, _ = state
            f = plsc.all_reduce_ffs(m_cur)
            sel = lane == f
            item = bcast(x, f)
            buk = lax.shift_right_logical(item - lo_own, 9)
            pos = plsc.load_gather(bk_cnt, [buk])
            m0 = sel & (pos < BCAP)
            s = buk * BCAP + pos
            plsc.store_scatter(bk_item, [s], item, mask=m0)
            plsc.store_scatter(
                bk_dst, [lax.shift_right_logical(s, 7), s & 127],
                lane + v * L, mask=m0)
            plsc.addupdate_scatter(bk_cnt, [buk], ones, mask=m0)
            return (m_cur & jnp.logical_not(sel), 0)

        lax.while_loop(cond, take, (m, 0))

    def route_body(u, carry):
        for j in range(4):
            route_one(u * 4 + j)
        return carry

    lax.fori_loop(0, NREQ // L // 4, route_body, 0)

    for j0 in range(0, NIDXR, 8):
        inv_copies = []
        for j in range(j0, j0 + 8):
            inv_copies.append(pltpu.async_copy(
                slotid_v.at[j], inv_hbm.at[bk_dst.at[j]], isem))
        for c in inv_copies:
            c.wait()

    def process_bucket(buf, buk, origin):
        nb = plsc.load_gather(bk_cnt, [jnp.zeros((L,), jnp.int32) + buk])
        bc = cnt_s[0]
        sbuf = lax.rem(bc, 2)

        @pl.when(bc >= 2)
        def _():
            pltpu.make_async_copy(
                gath_hbm.at[pl.ds(0, BCAP * DIM)],
                st_v.at[pl.ds(0, BCAP * DIM)],
                osem.at[sbuf]).wait()

        def vreg_body(v, carry):
            pv = v * L + lane
            m = pv < nb

            @pl.when(jnp.any(m))
            def _():
                x = bk_item[pl.ds(buk * BCAP + v * L, L)]
                lcs = jnp.where(m, x - origin, 0)
                dv = jnp.zeros((L,), jnp.int32)
                pidx = sbuf * (BCAP * DIM) + pv * DIM
                for _ in range(DIM):
                    g = plsc.load_gather(slab_v.at[buf], [dv, lcs], mask=m)
                    plsc.store_scatter(st_v, [pidx], g, mask=m)
                    dv = dv + 1
                    pidx = pidx + 1
            return carry

        lax.fori_loop(0, BCAP // L, vreg_body, 0)
        pltpu.async_copy(
            st_v.at[pl.ds(sbuf * (BCAP * DIM), BCAP * DIM)],
            gath_hbm.at[pl.ds((wid * SLOTPT + buk * BCAP) * DIM, BCAP * DIM)],
            osem.at[sbuf])
        cnt_s[0] = bc + 1

    def fire_slab(k, buf):
        pltpu.async_copy(tab_hbm.at[:, pl.ds(k * CW, CW)], slab_v.at[buf],
                         csem.at[buf])

    def wait_slab(buf):
        pltpu.make_async_copy(
            tab_hbm.at[:, pl.ds(0, CW)], slab_v.at[buf],
            csem.at[buf]).wait()

    nslab = cend - cstart

    def slab_body(i, carry):
        buf = lax.rem(i, 2)
        wait_slab(buf)
        process_bucket(buf, i, (cstart + i) * CW)

        @pl.when(i + 2 < nslab)
        def _():
            fire_slab(cstart + i + 2, buf)
        return carry

    lax.fori_loop(0, nslab, slab_body, 0)

    @pl.when(wid == NW - 1)
    def _():
        pltpu.sync_copy(tail_hbm, slab_v.at[0, :, pl.ds(0, 128)])
        process_bucket(0, nslab, TAILB)

    total = cnt_s[0]

    def drain_body(s, carry):
        @pl.when(s < total)
        def _():
            pltpu.make_async_copy(
                gath_hbm.at[pl.ds(0, BCAP * DIM)],
                st_v.at[pl.ds(0, BCAP * DIM)],
                osem.at[lax.rem(total + s, 2)]).wait()
        return carry

    lax.fori_loop(0, 2, drain_body, 0)


@functools.partial(
    pl.kernel,
    mesh=_mesh,
    out_type=jax.ShapeDtypeStruct((BATCH,), jnp.float32),
    compiler_params=pltpu.CompilerParams(
        needs_layout_passes=False, use_tc_tiling_on_sc=False),
    scratch_types=[
        pltpu.VMEM((4, 128), jnp.int32),
        pltpu.VMEM((4, 128), jnp.int32),
        pltpu.VMEM((BATCH // NW, DIM), jnp.float32),
        pltpu.VMEM((BATCH // NW, DIM), jnp.float32),
        pltpu.VMEM((BATCH // NW,), jnp.float32),
        pltpu.SemaphoreType.DMA,
    ],
)
def _sc_dot(gath_hbm, inv_hbm, out_hbm, iv_r, iv_c, rows_v, cols_v, out_v,
            sem):
    wid = lax.axis_index("s") * NC + lax.axis_index("c")
    bpw = BATCH // NW
    base = wid * bpw

    pltpu.sync_copy(inv_hbm.at[pl.ds(wid * 4, 4)], iv_r)
    pltpu.sync_copy(inv_hbm.at[pl.ds(BATCH // 128 + wid * 4, 4)], iv_c)

    copies = []
    for j in range(4):
        copies.append(pltpu.async_copy(
            gath_hbm.at[iv_r.at[j]], rows_v.at[pl.ds(j * 128, 128)], sem))
        copies.append(pltpu.async_copy(
            gath_hbm.at[iv_c.at[j]], cols_v.at[pl.ds(j * 128, 128)], sem))
    for c in copies:
        c.wait()

    lane = lax.iota(jnp.int32, L)

    def group_body(g, carry):
        row_ids = g * L + lane
        acc = jnp.zeros((L,), jnp.float32)
        dcol = jnp.zeros((L,), jnp.int32)
        for _ in range(DIM):
            r = plsc.load_gather(rows_v, [row_ids, dcol])
            c = plsc.load_gather(cols_v, [row_ids, dcol])
            acc = acc + r * c
            dcol = dcol + 1
        out_v[pl.ds(g * L, L)] = acc
        return carry

    lax.fori_loop(0, bpw // L, group_body, 0)

    pltpu.sync_copy(out_v, out_hbm.at[pl.ds(base, bpw)])


def kernel(rowIndex, colIndex, outEmbs):
    tabT = outEmbs.T
    tailT = lax.slice(outEmbs, (TAILB, 0), (NUM_ITEMS, DIM)).T
    gath, inv = _sc_harvest(rowIndex.astype(jnp.int32),
                            colIndex.astype(jnp.int32), tabT, tailT)
    return _sc_dot(gath.reshape(NSLOT, DIM), inv.reshape(INVPAD // 128, 128))


# speedup vs baseline: 79.9437x; 2.4260x over previous
import functools

import jax
import jax.numpy as jnp
from jax import lax
from jax.experimental import pallas as pl
from jax.experimental.pallas import tpu as pltpu
from jax.experimental.pallas import tpu_sc as plsc

NUM_ITEMS = 1000000
DIM = 64
BATCH = 16384

_info = plsc.get_sparse_core_info()
NC, NS, L = _info.num_cores, _info.num_subcores, _info.num_lanes
NW = NC * NS

NREQ = 2 * BATCH
CW = 512
NCH = (NUM_ITEMS - 64) // CW
TAIL0 = NCH * CW
TAILB = NUM_ITEMS - 128
NBUK = 64
BCAP = 80
SLOTPT = NBUK * BCAP

_mesh = plsc.VectorSubcoreMesh(core_axis_name="c", subcore_axis_name="s")
_IMIN = -2147483648


@functools.partial(
    pl.kernel,
    mesh=_mesh,
    out_type=jax.ShapeDtypeStruct((NREQ * DIM,), jnp.float32),
    compiler_params=pltpu.CompilerParams(
        needs_layout_passes=False, use_tc_tiling_on_sc=True),
    scratch_types=[
        pltpu.VMEM((NREQ,), jnp.int32),
        pltpu.VMEM((SLOTPT,), jnp.int32),
        pltpu.VMEM((SLOTPT,), jnp.int32),
        pltpu.VMEM((NBUK,), jnp.int32),
        pltpu.VMEM((2, DIM, CW), jnp.float32),
        pltpu.VMEM((2 * BCAP * DIM,), jnp.float32),
        pltpu.SMEM((4,), jnp.int32),
        pltpu.SemaphoreType.DMA((2,)),
        pltpu.SemaphoreType.DMA((2,)),
    ],
)
def _sc_harvest(row_hbm, col_hbm, tab_hbm, tail_hbm, gath_hbm,
                req_v, bk_item, bk_dst, bk_cnt, slab_v, st_v, cnt_s,
                csem, osem):
    wid = lax.axis_index("s") * NC + lax.axis_index("c")
    lane = lax.iota(jnp.int32, L)
    ones = jnp.zeros((L,), jnp.int32) + 1

    cstart = (NCH * wid) // NW
    cend = (NCH * (wid + 1)) // NW
    lo_own = cstart * CW
    hi_own = jnp.where(wid == NW - 1, NUM_ITEMS, cend * CW)

    pltpu.async_copy(tab_hbm.at[:, pl.ds(cstart * CW, CW)], slab_v.at[0],
                     csem.at[0])
    pltpu.async_copy(tab_hbm.at[:, pl.ds((cstart + 1) * CW, CW)], slab_v.at[1],
                     csem.at[1])

    pltpu.sync_copy(row_hbm, req_v.at[pl.ds(0, BATCH)])
    pltpu.sync_copy(col_hbm, req_v.at[pl.ds(BATCH, BATCH)])

    def zero_body(v, carry):
        bk_cnt[pl.ds(v * L, L)] = jnp.zeros((L,), jnp.int32)
        return carry
    lax.fori_loop(0, NBUK // L, zero_body, 0)
    cnt_s[0] = 0
    cnt_s[2] = 0
    cnt_s[3] = 0

    def bcast(vec, f_splat):
        return jnp.take_along_axis(vec, f_splat, axis=0,
                                   mode="promise_in_bounds")

    def route_one(v):
        x = req_v[pl.ds(v * L, L)]
        m = (x >= lo_own) & (x < hi_own)

        def cond(state):
            return jnp.any(state[0])

        def take(state):
            m_cur, _ = state
            f = plsc.all_reduce_ffs(m_cur)
            sel = lane == f
            item = bcast(x, f)
            buk = lax.shift_right_logical(item - lo_own, 9)
            pos = plsc.load_gather(bk_cnt, [buk])
            m0 = sel & (pos < BCAP)
            s = buk * BCAP + pos
            plsc.store_scatter(bk_item, [s], item, mask=m0)
            plsc.store_scatter(bk_dst, [s], lane + v * L, mask=m0)
            plsc.addupdate_scatter(bk_cnt, [buk], ones, mask=m0)
            return (m_cur & jnp.logical_not(sel), 0)

        lax.while_loop(cond, take, (m, 0))

    def route_body(u, carry):
        for j in range(4):
            route_one(u * 4 + j)
        return carry

    lax.fori_loop(0, NREQ // L // 4, route_body, 0)

    def process_bucket(buf, buk, origin):
        nb = plsc.load_gather(bk_cnt, [jnp.zeros((L,), jnp.int32) + buk])
        bc = cnt_s[0]
        sbuf = lax.rem(bc, 2)

        nprev = jnp.where(sbuf == 0, cnt_s[2], cnt_s[3])

        def drain_one(s, carry):
            pltpu.make_async_copy(
                gath_hbm.at[pl.ds(0, DIM)],
                st_v.at[pl.ds(0, DIM)],
                osem.at[sbuf]).wait()
            return carry
        lax.fori_loop(0, nprev, drain_one, 0)

        def vreg_body(v, carry):
            pv = v * L + lane
            m = pv < nb

            @pl.when(jnp.any(m))
            def _():
                x = bk_item[pl.ds(buk * BCAP + v * L, L)]
                lcs = jnp.where(m, x - origin, 0)
                dv = jnp.zeros((L,), jnp.int32)
                pidx = sbuf * (BCAP * DIM) + pv * DIM
                for _ in range(DIM):
                    g = plsc.load_gather(slab_v.at[buf], [dv, lcs], mask=m)
                    plsc.store_scatter(st_v, [pidx], g, mask=m)
                    dv = dv + 1
                    pidx = pidx + 1

                dvec = bk_dst[pl.ds(buk * BCAP + v * L, L)]
                enc = dvec * 128 + pv

                def cond(state):
                    return jnp.any(state[0])

                def fire(state):
                    m_cur, _ = state
                    f = plsc.all_reduce_ffs(m_cur)
                    sel = lane == f
                    e = jnp.max(jnp.where(sel, enc, jnp.int32(_IMIN)))
                    dst = lax.shift_right_logical(e, 7)
                    stoff = e & 127
                    pltpu.async_copy(
                        st_v.at[pl.ds(sbuf * (BCAP * DIM) + stoff * DIM, DIM)],
                        gath_hbm.at[pl.ds(dst * DIM, DIM)],
                        osem.at[sbuf])
                    return (m_cur & jnp.logical_not(sel), 0)

                lax.while_loop(cond, fire, (m, 0))
            return carry

        lax.fori_loop(0, BCAP // L, vreg_body, 0)

        nfired = jnp.minimum(jnp.max(nb), BCAP)

        @pl.when(sbuf == 0)
        def _():
            cnt_s[2] = nfired

        @pl.when(sbuf == 1)
        def _():
            cnt_s[3] = nfired
        cnt_s[0] = bc + 1

    def fire_slab(k, buf):
        pltpu.async_copy(tab_hbm.at[:, pl.ds(k * CW, CW)], slab_v.at[buf],
                         csem.at[buf])

    def wait_slab(buf):
        pltpu.make_async_copy(
            tab_hbm.at[:, pl.ds(0, CW)], slab_v.at[buf],
            csem.at[buf]).wait()

    nslab = cend - cstart

    def slab_body(i, carry):
        buf = lax.rem(i, 2)
        wait_slab(buf)
        process_bucket(buf, i, (cstart + i) * CW)

        @pl.when(i + 2 < nslab)
        def _():
            fire_slab(cstart + i + 2, buf)
        return carry

    lax.fori_loop(0, nslab, slab_body, 0)

    @pl.when(wid == NW - 1)
    def _():
        pltpu.sync_copy(tail_hbm, slab_v.at[0, :, pl.ds(0, 128)])
        process_bucket(0, nslab, TAILB)

    for b in range(2):
        def drain_body(s, carry):
            pltpu.make_async_copy(
                gath_hbm.at[pl.ds(0, DIM)],
                st_v.at[pl.ds(0, DIM)],
                osem.at[b]).wait()
            return carry
        lax.fori_loop(0, cnt_s[2 + b], drain_body, 0)


@functools.partial(
    pl.kernel,
    mesh=_mesh,
    out_type=jax.ShapeDtypeStruct((BATCH,), jnp.float32),
    compiler_params=pltpu.CompilerParams(
        needs_layout_passes=False, use_tc_tiling_on_sc=False),
    scratch_types=[
        pltpu.VMEM((BATCH // NW, DIM), jnp.float32),
        pltpu.VMEM((BATCH // NW, DIM), jnp.float32),
        pltpu.VMEM((BATCH // NW,), jnp.float32),
        pltpu.SemaphoreType.DMA,
    ],
)
def _sc_dot(gath_hbm, out_hbm, rows_v, cols_v, out_v, sem):
    wid = lax.axis_index("s") * NC + lax.axis_index("c")
    bpw = BATCH // NW
    base = wid * bpw
    c1 = pltpu.async_copy(gath_hbm.at[pl.ds(base, bpw)], rows_v, sem)
    c2 = pltpu.async_copy(gath_hbm.at[pl.ds(BATCH + base, bpw)], cols_v, sem)
    c1.wait()
    c2.wait()

    lane = lax.iota(jnp.int32, L)

    def group_body(g, carry):
        row_ids = g * L + lane
        acc = jnp.zeros((L,), jnp.float32)
        dcol = jnp.zeros((L,), jnp.int32)
        for _ in range(DIM):
            r = plsc.load_gather(rows_v, [row_ids, dcol])
            c = plsc.load_gather(cols_v, [row_ids, dcol])
            acc = acc + r * c
            dcol = dcol + 1
        out_v[pl.ds(g * L, L)] = acc
        return carry

    lax.fori_loop(0, bpw // L, group_body, 0)

    pltpu.sync_copy(out_v, out_hbm.at[pl.ds(base, bpw)])


def kernel(rowIndex, colIndex, outEmbs):
    tabT = outEmbs.T
    tailT = lax.slice(outEmbs, (TAILB, 0), (NUM_ITEMS, DIM)).T
    gath = _sc_harvest(rowIndex.astype(jnp.int32), colIndex.astype(jnp.int32),
                       tabT, tailT)
    return _sc_dot(gath.reshape(NREQ, DIM))


# speedup vs baseline: 80.8126x; 1.0109x over previous
import functools

import jax
import jax.numpy as jnp
from jax import lax
from jax.experimental import pallas as pl
from jax.experimental.pallas import tpu as pltpu
from jax.experimental.pallas import tpu_sc as plsc

NUM_ITEMS = 1000000
DIM = 64
BATCH = 16384

_info = plsc.get_sparse_core_info()
NC, NS, L = _info.num_cores, _info.num_subcores, _info.num_lanes
NW = NC * NS

NREQ = 2 * BATCH
CW = 512
NCH = (NUM_ITEMS - 64) // CW
TAIL0 = NCH * CW
TAILB = NUM_ITEMS - 128
NBUK = 64
BCAP = 96
NST = 8

_mesh = plsc.VectorSubcoreMesh(core_axis_name="c", subcore_axis_name="s")
_IMIN = -2147483648


@functools.partial(
    pl.kernel,
    mesh=_mesh,
    out_type=jax.ShapeDtypeStruct((NREQ * DIM,), jnp.float32),
    compiler_params=pltpu.CompilerParams(
        needs_layout_passes=False, use_tc_tiling_on_sc=True),
    scratch_types=[
        pltpu.VMEM((NREQ,), jnp.int32),
        pltpu.VMEM((NBUK * BCAP,), jnp.int32),
        pltpu.VMEM((NBUK * BCAP,), jnp.int32),
        pltpu.VMEM((NBUK,), jnp.int32),
        pltpu.VMEM((2, DIM, CW), jnp.float32),
        pltpu.VMEM((NST * DIM,), jnp.float32),
        pltpu.SMEM((4,), jnp.int32),
        pltpu.SemaphoreType.DMA((2,)),
        pltpu.SemaphoreType.DMA((NST,)),
    ],
)
def _sc_harvest(row_hbm, col_hbm, tab_hbm, tail_hbm, out_hbm,
                req_v, bk_item, bk_dst, bk_cnt, slab_v, st_v, cnt_s,
                csem, ssem):
    wid = lax.axis_index("s") * NC + lax.axis_index("c")
    lane = lax.iota(jnp.int32, L)
    ones = jnp.zeros((L,), jnp.int32) + 1

    cstart = (NCH * wid) // NW
    cend = (NCH * (wid + 1)) // NW
    lo_own = cstart * CW
    hi_own = jnp.where(wid == NW - 1, NUM_ITEMS, cend * CW)

    pltpu.async_copy(tab_hbm.at[:, pl.ds(cstart * CW, CW)], slab_v.at[0],
                     csem.at[0])
    pltpu.async_copy(tab_hbm.at[:, pl.ds((cstart + 1) * CW, CW)], slab_v.at[1],
                     csem.at[1])

    pltpu.sync_copy(row_hbm, req_v.at[pl.ds(0, BATCH)])
    pltpu.sync_copy(col_hbm, req_v.at[pl.ds(BATCH, BATCH)])

    def zero_body(v, carry):
        bk_cnt[pl.ds(v * L, L)] = jnp.zeros((L,), jnp.int32)
        return carry
    lax.fori_loop(0, NBUK // L, zero_body, 0)
    cnt_s[0] = 0

    def bcast(vec, f_splat):
        return jnp.take_along_axis(vec, f_splat, axis=0,
                                   mode="promise_in_bounds")

    def route_one(v):
        x = req_v[pl.ds(v * L, L)]
        m = (x >= lo_own) & (x < hi_own)

        def cond(state):
            return jnp.any(state[0])

        def take(state):
            m_cur, _ = state
            f = plsc.all_reduce_ffs(m_cur)
            sel = lane == f
            item = bcast(x, f)
            buk = lax.shift_right_logical(item - lo_own, 9)
            pos = plsc.load_gather(bk_cnt, [buk])
            m0 = sel & (pos < BCAP)
            slot = buk * BCAP + pos
            plsc.store_scatter(bk_item, [slot], item, mask=m0)
            plsc.store_scatter(bk_dst, [slot], lane + v * L, mask=m0)
            plsc.addupdate_scatter(bk_cnt, [buk], ones, mask=m0)
            return (m_cur & jnp.logical_not(sel), 0)

        lax.while_loop(cond, take, (m, 0))

    def route_body(u, carry):
        for j in range(4):
            route_one(u * 4 + j)
        return carry

    lax.fori_loop(0, NREQ // L // 4, route_body, 0)

    dimq = [lane + q * L for q in range(DIM // L)]

    def process_bucket(buf, buk, origin):
        nb = plsc.load_gather(bk_cnt, [jnp.zeros((L,), jnp.int32) + buk])

        def scan_body(v, carry):
            base = buk * BCAP + v * L
            x = bk_item[pl.ds(base, L)]
            d = bk_dst[pl.ds(base, L)]
            m = (v * L + lane) < nb

            def cond(state):
                return jnp.any(state[0])

            def take(state):
                m_cur, _ = state
                f = plsc.all_reduce_ffs(m_cur)
                sel = lane == f
                lc = bcast(x, f) - origin
                dst = jnp.max(jnp.where(sel, d, jnp.int32(_IMIN)))
                mc = cnt_s[0]
                slot = lax.rem(mc, NST)

                @pl.when(mc >= NST)
                def _():
                    pltpu.make_async_copy(
                        out_hbm.at[pl.ds(0, DIM)],
                        st_v.at[pl.ds(0, DIM)],
                        ssem.at[slot]).wait()
                for q in range(DIM // L):
                    g = plsc.load_gather(slab_v.at[buf], [dimq[q], lc])
                    st_v[pl.ds(slot * DIM + q * L, L)] = g
                pltpu.async_copy(
                    st_v.at[pl.ds(slot * DIM, DIM)],
                    out_hbm.at[pl.ds(dst * DIM, DIM)],
                    ssem.at[slot])
                cnt_s[0] = mc + 1
                return (m_cur & jnp.logical_not(sel), 0)

            lax.while_loop(cond, take, (m, 0))
            return carry

        lax.fori_loop(0, BCAP // L, scan_body, 0)

    def fire_slab(k, buf):
        pltpu.async_copy(tab_hbm.at[:, pl.ds(k * CW, CW)], slab_v.at[buf],
                         csem.at[buf])

    def wait_slab(buf):
        pltpu.make_async_copy(
            tab_hbm.at[:, pl.ds(0, CW)], slab_v.at[buf],
            csem.at[buf]).wait()

    nslab = cend - cstart

    def slab_body(i, carry):
        buf = lax.rem(i, 2)
        wait_slab(buf)
        process_bucket(buf, i, (cstart + i) * CW)

        @pl.when(i + 2 < nslab)
        def _():
            fire_slab(cstart + i + 2, buf)
        return carry

    lax.fori_loop(0, nslab, slab_body, 0)

    @pl.when(wid == NW - 1)
    def _():
        pltpu.sync_copy(tail_hbm, slab_v.at[0, :, pl.ds(0, 128)])
        process_bucket(0, nslab, TAILB)

    total = cnt_s[0]

    def drain_body(s, carry):
        @pl.when(s < total)
        def _():
            pltpu.make_async_copy(
                out_hbm.at[pl.ds(0, DIM)],
                st_v.at[pl.ds(0, DIM)],
                ssem.at[lax.rem(s, NST)]).wait()
        return carry

    lax.fori_loop(0, NST, drain_body, 0)


@functools.partial(
    pl.kernel,
    mesh=_mesh,
    out_type=jax.ShapeDtypeStruct((BATCH,), jnp.float32),
    compiler_params=pltpu.CompilerParams(
        needs_layout_passes=False, use_tc_tiling_on_sc=False),
    scratch_types=[
        pltpu.VMEM((BATCH // NW, DIM), jnp.float32),
        pltpu.VMEM((BATCH // NW, DIM), jnp.float32),
        pltpu.VMEM((BATCH // NW,), jnp.float32),
        pltpu.SemaphoreType.DMA,
    ],
)
def _sc_dot(gath_hbm, out_hbm, rows_v, cols_v, out_v, sem):
    wid = lax.axis_index("s") * NC + lax.axis_index("c")
    bpw = BATCH // NW
    base = wid * bpw
    c1 = pltpu.async_copy(gath_hbm.at[pl.ds(base, bpw)], rows_v, sem)
    c2 = pltpu.async_copy(gath_hbm.at[pl.ds(BATCH + base, bpw)], cols_v, sem)
    c1.wait()
    c2.wait()

    lane = lax.iota(jnp.int32, L)

    def group_body(g, carry):
        row_ids = g * L + lane
        acc = jnp.zeros((L,), jnp.float32)
        dcol = jnp.zeros((L,), jnp.int32)
        for _ in range(DIM):
            r = plsc.load_gather(rows_v, [row_ids, dcol])
            c = plsc.load_gather(cols_v, [row_ids, dcol])
            acc = acc + r * c
            dcol = dcol + 1
        out_v[pl.ds(g * L, L)] = acc
        return carry

    lax.fori_loop(0, BATCH // NW // L, group_body, 0)

    pltpu.sync_copy(out_v, out_hbm.at[pl.ds(base, bpw)])


def kernel(rowIndex, colIndex, outEmbs):
    tabT = outEmbs.T
    tailT = lax.slice(outEmbs, (TAILB, 0), (NUM_ITEMS, DIM)).T
    gath = _sc_harvest(rowIndex.astype(jnp.int32), colIndex.astype(jnp.int32),
                       tabT, tailT)
    return _sc_dot(gath.reshape(NREQ, DIM))
